# Initial kernel scaffold; baseline (speedup 1.0000x reference)
import jax
import jax.numpy as jnp
from jax.experimental import pallas as pl


def kernel(property_offset, node_type, segment_ids):
    raise NotImplementedError("write your pallas kernel here")


# speedup vs baseline: 11.5998x; 11.5998x over previous
import functools

import jax
import jax.numpy as jnp
from jax import lax
from jax.experimental import pallas as pl
from jax.experimental.pallas import tpu as pltpu
from jax.experimental.pallas import tpu_sc as plsc

L = 16
NW = 32
CHUNK = 3200
NSEG = 1024
SEG_PAD = 1056
PO_PAD = 96


def _sc_body(po_hbm, nt_hbm, seg_hbm, out_hbm, po_v, nt_v, seg_v, s_v, e_v):
    c = lax.axis_index("c")
    s = lax.axis_index("s")
    wid = s * 2 + c
    base = wid * CHUNK

    pltpu.sync_copy(po_hbm, po_v)
    pltpu.sync_copy(nt_hbm.at[pl.ds(base, CHUNK)], nt_v)
    pltpu.sync_copy(seg_hbm.at[pl.ds(base, CHUNK)], seg_v.at[pl.ds(L, CHUNK)])
    seg_v[pl.ds(0, L)] = jnp.full((L,), -1, jnp.int32)
    seg_v[pl.ds(L + CHUNK, L)] = jnp.full((L,), 2**30, jnp.int32)

    zeros = jnp.zeros((L,), jnp.float32)

    def zero_body(j, carry):
        s_v[pl.ds(j * L, L)] = zeros
        e_v[pl.ds(j * L, L)] = zeros
        return carry

    lax.fori_loop(0, SEG_PAD // L, zero_body, 0)

    lanes = lax.iota(jnp.int32, L)

    def body(i, carry):
        off = L + i * L
        seg = seg_v[pl.ds(off, L)]
        pos = lanes + off
        seg_prev = plsc.load_gather(seg_v, [pos - 1])
        seg_next = plsc.load_gather(seg_v, [pos + 1])
        nt = nt_v[pl.ds(i * L, L)]
        v = plsc.load_gather(po_v, [nt])
        cum = plsc.cumsum(v)
        c_incl = cum + carry
        c_excl = c_incl - v
        plsc.store_scatter(s_v, [seg], c_excl, mask=seg != seg_prev)
        plsc.store_scatter(e_v, [seg], c_incl, mask=seg != seg_next)
        return carry + jnp.sum(v)

    lax.fori_loop(0, CHUNK // L, body, jnp.float32(0.0))

    def diff_body(j, carry):
        s_v[pl.ds(j * L, L)] = e_v[pl.ds(j * L, L)] - s_v[pl.ds(j * L, L)]
        return carry

    lax.fori_loop(0, NSEG // L, diff_body, 0)

    pltpu.sync_copy(s_v.at[pl.ds(0, NSEG)], out_hbm.at[wid])


_sc_partials = pl.kernel(
    _sc_body,
    out_type=jax.ShapeDtypeStruct((NW, NSEG), jnp.float32),
    mesh=plsc.VectorSubcoreMesh(
        core_axis_name="c", subcore_axis_name="s", num_cores=2, num_subcores=16),
    compiler_params=pltpu.CompilerParams(needs_layout_passes=False),
    scratch_types=[
        pltpu.VMEM((PO_PAD,), jnp.float32),
        pltpu.VMEM((CHUNK,), jnp.int32),
        pltpu.VMEM((CHUNK + 2 * L,), jnp.int32),
        pltpu.VMEM((SEG_PAD,), jnp.float32),
        pltpu.VMEM((SEG_PAD,), jnp.float32),
    ],
)


def _reduce_body(in_ref, out_ref):
    out_ref[...] = jnp.sum(in_ref[...], axis=0)


_tc_reduce = pl.pallas_call(
    _reduce_body,
    out_shape=jax.ShapeDtypeStruct((NSEG,), jnp.float32),
)


def kernel(property_offset, node_type, segment_ids):
    nz = property_offset.shape[0]
    n = node_type.shape[0]
    total = NW * CHUNK
    po = jnp.zeros((PO_PAD,), jnp.float32).at[:nz].set(property_offset)
    nt = jnp.concatenate(
        [node_type.astype(jnp.int32), jnp.full((total - n,), nz, jnp.int32)])
    seg = jnp.concatenate(
        [segment_ids.astype(jnp.int32), jnp.full((total - n,), NSEG, jnp.int32)])
    partials = _sc_partials(po, nt, seg)
    return _tc_reduce(partials)


# speedup vs baseline: 12.1123x; 1.0442x over previous
import jax
import jax.numpy as jnp
from jax import lax
from jax.experimental import pallas as pl
from jax.experimental.pallas import tpu as pltpu
from jax.experimental.pallas import tpu_sc as plsc

L = 16
NW = 32
N = 100000
CHUNK = 3136
LAST = N - (NW - 1) * CHUNK
NSEG = 1024
SEG_PAD = 1040


def _sc_body(po_hbm, nt_hbm, seg_hbm, out_hbm, po_v, nt_v, seg_v, s_v, e_v):
    c = lax.axis_index("c")
    s = lax.axis_index("s")
    wid = s * 2 + c
    base = wid * CHUNK
    is_last = wid == NW - 1
    nvec = jnp.where(is_last, LAST // L, CHUNK // L)

    pltpu.sync_copy(po_hbm, po_v)

    @pl.when(jnp.logical_not(is_last))
    def _():
        pltpu.sync_copy(nt_hbm.at[pl.ds(base, CHUNK)], nt_v)
        pltpu.sync_copy(seg_hbm.at[pl.ds(base, CHUNK)],
                        seg_v.at[pl.ds(L, CHUNK)])

    @pl.when(is_last)
    def _():
        pltpu.sync_copy(nt_hbm.at[pl.ds(base, LAST)], nt_v.at[pl.ds(0, LAST)])
        pltpu.sync_copy(seg_hbm.at[pl.ds(base, LAST)],
                        seg_v.at[pl.ds(L, LAST)])

    seg_v[pl.ds(0, L)] = jnp.full((L,), -1, jnp.int32)
    seg_v[pl.ds(L + nvec * L, L)] = jnp.full((L,), 2**30, jnp.int32)

    zeros = jnp.zeros((L,), jnp.float32)

    def zero_body(j, carry):
        s_v[pl.ds(j * L, L)] = zeros
        e_v[pl.ds(j * L, L)] = zeros
        return carry

    lax.fori_loop(0, SEG_PAD // L, zero_body, 0)

    def body(i, carry):
        off = L + i * L
        seg = seg_v[pl.ds(off, L)]
        seg_prev = seg_v[pl.ds(off - 1, L)]
        seg_next = seg_v[pl.ds(off + 1, L)]
        nt = nt_v[pl.ds(i * L, L)]
        v = plsc.load_gather(po_v, [nt])
        cum = plsc.cumsum(v)
        c_incl = cum + carry
        c_excl = c_incl - v
        plsc.store_scatter(s_v, [seg], c_excl, mask=seg != seg_prev)
        plsc.store_scatter(e_v, [seg], c_incl, mask=seg != seg_next)
        return carry + lax.squeeze(lax.slice(cum, (L - 1,), (L,)), (0,))

    lax.fori_loop(0, nvec, body, jnp.float32(0.0))

    def diff_body(j, carry):
        s_v[pl.ds(j * L, L)] = e_v[pl.ds(j * L, L)] - s_v[pl.ds(j * L, L)]
        return carry

    lax.fori_loop(0, NSEG // L, diff_body, 0)

    pltpu.sync_copy(s_v.at[pl.ds(0, NSEG)], out_hbm.at[wid])


_sc_partials = pl.kernel(
    _sc_body,
    out_type=jax.ShapeDtypeStruct((NW, NSEG), jnp.float32),
    mesh=plsc.VectorSubcoreMesh(
        core_axis_name="c", subcore_axis_name="s", num_cores=2, num_subcores=16),
    compiler_params=pltpu.CompilerParams(needs_layout_passes=False),
    scratch_types=[
        pltpu.VMEM((96,), jnp.float32),
        pltpu.VMEM((CHUNK,), jnp.int32),
        pltpu.VMEM((CHUNK + 2 * L,), jnp.int32),
        pltpu.VMEM((SEG_PAD,), jnp.float32),
        pltpu.VMEM((SEG_PAD,), jnp.float32),
    ],
)


def _reduce_body(in_ref, out_ref):
    out_ref[...] = jnp.sum(in_ref[...], axis=0)


_tc_reduce = pl.pallas_call(
    _reduce_body,
    out_shape=jax.ShapeDtypeStruct((NSEG,), jnp.float32),
)


def kernel(property_offset, node_type, segment_ids):
    po = jnp.zeros((96,), jnp.float32).at[:property_offset.shape[0]].set(
        property_offset)
    partials = _sc_partials(po, node_type.astype(jnp.int32),
                            segment_ids.astype(jnp.int32))
    return _tc_reduce(partials)


# speedup vs baseline: 13.3945x; 1.1059x over previous
import jax
import jax.numpy as jnp
from jax import lax
from jax.experimental import pallas as pl
from jax.experimental.pallas import tpu as pltpu
from jax.experimental.pallas import tpu_sc as plsc

L = 16
NW = 32
N = 100000
CHUNK = 3136
LAST = N - (NW - 1) * CHUNK
NSEG = 1024
SEG_PAD = 1040
SENT = 1032


def _sc_body(po_hbm, nt_hbm, seg_hbm, out_hbm, po_v, nt_v, seg_v, s_v, e_v):
    c = lax.axis_index("c")
    s = lax.axis_index("s")
    wid = s * 2 + c
    base = wid * CHUNK
    is_last = wid == NW - 1
    nvec = jnp.where(is_last, LAST // L, CHUNK // L)

    pltpu.sync_copy(po_hbm, po_v)

    @pl.when(jnp.logical_not(is_last))
    def _():
        pltpu.sync_copy(nt_hbm.at[pl.ds(base, CHUNK)], nt_v)
        pltpu.sync_copy(seg_hbm.at[pl.ds(base, CHUNK)],
                        seg_v.at[pl.ds(0, CHUNK)])

    @pl.when(is_last)
    def _():
        pltpu.sync_copy(nt_hbm.at[pl.ds(base, LAST)], nt_v.at[pl.ds(0, LAST)])
        pltpu.sync_copy(seg_hbm.at[pl.ds(base, LAST)],
                        seg_v.at[pl.ds(0, LAST)])

    seg_v[pl.ds(nvec * L, L)] = jnp.full((L,), SENT, jnp.int32)

    zeros = jnp.zeros((L,), jnp.float32)

    def zero_body(j, carry):
        s_v[pl.ds(j * L, L)] = zeros
        e_v[pl.ds(j * L, L)] = zeros
        return carry

    lax.fori_loop(0, SEG_PAD // L, zero_body, 0)

    @plsc.parallel_loop(0, nvec, unroll=4, carry=jnp.float32(0.0))
    def _(i, carry):
        off = i * L
        seg = seg_v[pl.ds(off, L)]
        seg_next = seg_v[pl.ds(off + 1, L)]
        nt = nt_v[pl.ds(off, L)]
        v = plsc.load_gather(po_v, [nt])
        c_incl = plsc.cumsum(v) + carry
        m_end = seg != seg_next
        plsc.store_scatter(e_v, [seg], c_incl, mask=m_end)
        plsc.store_scatter(s_v, [seg_next], c_incl, mask=m_end)
        return lax.squeeze(lax.slice(c_incl, (L - 1,), (L,)), (0,))

    def diff_body(j, carry):
        s_v[pl.ds(j * L, L)] = e_v[pl.ds(j * L, L)] - s_v[pl.ds(j * L, L)]
        return carry

    lax.fori_loop(0, NSEG // L, diff_body, 0)

    pltpu.sync_copy(s_v.at[pl.ds(0, NSEG)], out_hbm.at[wid])


_sc_partials = pl.kernel(
    _sc_body,
    out_type=jax.ShapeDtypeStruct((NW, NSEG), jnp.float32),
    mesh=plsc.VectorSubcoreMesh(
        core_axis_name="c", subcore_axis_name="s", num_cores=2, num_subcores=16),
    compiler_params=pltpu.CompilerParams(needs_layout_passes=False),
    scratch_types=[
        pltpu.VMEM((96,), jnp.float32),
        pltpu.VMEM((CHUNK,), jnp.int32),
        pltpu.VMEM((CHUNK + L,), jnp.int32),
        pltpu.VMEM((SEG_PAD,), jnp.float32),
        pltpu.VMEM((SEG_PAD,), jnp.float32),
    ],
)


def _reduce_body(in_ref, out_ref):
    out_ref[...] = jnp.sum(in_ref[...], axis=0)


_tc_reduce = pl.pallas_call(
    _reduce_body,
    out_shape=jax.ShapeDtypeStruct((NSEG,), jnp.float32),
)


def kernel(property_offset, node_type, segment_ids):
    po = jnp.zeros((96,), jnp.float32).at[:property_offset.shape[0]].set(
        property_offset)
    partials = _sc_partials(po, node_type.astype(jnp.int32),
                            segment_ids.astype(jnp.int32))
    return _tc_reduce(partials)


# speedup vs baseline: 13.4962x; 1.0076x over previous
import jax
import jax.numpy as jnp
from jax import lax
from jax.experimental import pallas as pl
from jax.experimental.pallas import tpu as pltpu
from jax.experimental.pallas import tpu_sc as plsc

L = 16
NW = 32
N = 100000
CHUNK = 3136
LAST = N - (NW - 1) * CHUNK
NSEG = 1024
SEG_PAD = 1040
SENT = 1032


def _sc_body(po_hbm, nt_hbm, seg_hbm, out_hbm, po_v, nt_v, seg_v, s_v, e_v):
    c = lax.axis_index("c")
    s = lax.axis_index("s")
    wid = s * 2 + c
    base = wid * CHUNK
    is_last = wid == NW - 1
    nvec = jnp.where(is_last, LAST // L, CHUNK // L)

    pltpu.sync_copy(po_hbm, po_v)

    @pl.when(jnp.logical_not(is_last))
    def _():
        pltpu.sync_copy(nt_hbm.at[pl.ds(base, CHUNK)], nt_v)
        pltpu.sync_copy(seg_hbm.at[pl.ds(base, CHUNK)],
                        seg_v.at[pl.ds(0, CHUNK)])

    @pl.when(is_last)
    def _():
        pltpu.sync_copy(nt_hbm.at[pl.ds(base, LAST)], nt_v.at[pl.ds(0, LAST)])
        pltpu.sync_copy(seg_hbm.at[pl.ds(base, LAST)],
                        seg_v.at[pl.ds(0, LAST)])

    seg_v[pl.ds(nvec * L, L)] = jnp.full((L,), SENT, jnp.int32)

    zeros = jnp.zeros((L,), jnp.float32)

    def zero_body(j, carry):
        s_v[pl.ds(j * L, L)] = zeros
        e_v[pl.ds(j * L, L)] = zeros
        return carry

    lax.fori_loop(0, SEG_PAD // L, zero_body, 0)

    @plsc.parallel_loop(0, nvec, unroll=8, carry=jnp.float32(0.0))
    def _(i, carry):
        off = i * L
        seg = seg_v[pl.ds(off, L)]
        seg_next = seg_v[pl.ds(off + 1, L)]
        nt = nt_v[pl.ds(off, L)]
        v = plsc.load_gather(po_v, [nt])
        c_incl = plsc.cumsum(v) + carry
        m_end = seg != seg_next
        plsc.store_scatter(e_v, [seg], c_incl, mask=m_end)
        plsc.store_scatter(s_v, [seg_next], c_incl, mask=m_end)
        return lax.squeeze(lax.slice(c_incl, (L - 1,), (L,)), (0,))

    def diff_body(j, carry):
        s_v[pl.ds(j * L, L)] = e_v[pl.ds(j * L, L)] - s_v[pl.ds(j * L, L)]
        return carry

    lax.fori_loop(0, NSEG // L, diff_body, 0)

    pltpu.sync_copy(s_v.at[pl.ds(0, NSEG)], out_hbm.at[wid])


_sc_partials = pl.kernel(
    _sc_body,
    out_type=jax.ShapeDtypeStruct((NW, NSEG), jnp.float32),
    mesh=plsc.VectorSubcoreMesh(
        core_axis_name="c", subcore_axis_name="s", num_cores=2, num_subcores=16),
    compiler_params=pltpu.CompilerParams(needs_layout_passes=False),
    scratch_types=[
        pltpu.VMEM((96,), jnp.float32),
        pltpu.VMEM((CHUNK,), jnp.int32),
        pltpu.VMEM((CHUNK + L,), jnp.int32),
        pltpu.VMEM((SEG_PAD,), jnp.float32),
        pltpu.VMEM((SEG_PAD,), jnp.float32),
    ],
)


def _reduce_body(in_ref, out_ref):
    out_ref[...] = jnp.sum(in_ref[...], axis=0)


_tc_reduce = pl.pallas_call(
    _reduce_body,
    out_shape=jax.ShapeDtypeStruct((NSEG,), jnp.float32),
)


def kernel(property_offset, node_type, segment_ids):
    po = jnp.zeros((96,), jnp.float32).at[:property_offset.shape[0]].set(
        property_offset)
    partials = _sc_partials(po, node_type.astype(jnp.int32),
                            segment_ids.astype(jnp.int32))
    return _tc_reduce(partials)


# speedup vs baseline: 13.6248x; 1.0095x over previous
import jax
import jax.numpy as jnp
from jax import lax
from jax.experimental import pallas as pl
from jax.experimental.pallas import tpu as pltpu
from jax.experimental.pallas import tpu_sc as plsc

L = 16
NW = 32
N = 100000
CHUNK = 3136
LAST = N - (NW - 1) * CHUNK
NSEG = 1024
SEG_PAD = 1040
SENT = 1032
MAX_Z = 89


def _sc_body(po_hbm, nt_hbm, seg_hbm, out_hbm, po_v, nt_v, seg_v, s_v, e_v):
    c = lax.axis_index("c")
    s = lax.axis_index("s")
    wid = s * 2 + c
    base = wid * CHUNK
    is_last = wid == NW - 1
    nvec = jnp.where(is_last, LAST // L, CHUNK // L)

    pltpu.sync_copy(po_hbm, po_v.at[pl.ds(0, MAX_Z)])

    @pl.when(jnp.logical_not(is_last))
    def _():
        pltpu.sync_copy(nt_hbm.at[pl.ds(base, CHUNK)], nt_v)
        pltpu.sync_copy(seg_hbm.at[pl.ds(base, CHUNK)],
                        seg_v.at[pl.ds(0, CHUNK)])

    @pl.when(is_last)
    def _():
        pltpu.sync_copy(nt_hbm.at[pl.ds(base, LAST)], nt_v.at[pl.ds(0, LAST)])
        pltpu.sync_copy(seg_hbm.at[pl.ds(base, LAST)],
                        seg_v.at[pl.ds(0, LAST)])

    seg_v[pl.ds(nvec * L, L)] = jnp.full((L,), SENT, jnp.int32)

    zeros = jnp.zeros((L,), jnp.float32)

    def zero_body(j, carry):
        s_v[pl.ds(j * L, L)] = zeros
        e_v[pl.ds(j * L, L)] = zeros
        return carry

    lax.fori_loop(0, SEG_PAD // L, zero_body, 0)

    @plsc.parallel_loop(0, nvec, unroll=8, carry=jnp.float32(0.0))
    def _(i, carry):
        off = i * L
        seg = seg_v[pl.ds(off, L)]
        seg_next = seg_v[pl.ds(off + 1, L)]
        nt = nt_v[pl.ds(off, L)]
        v = plsc.load_gather(po_v, [nt])
        c_incl = plsc.cumsum(v) + carry
        m_end = seg != seg_next
        plsc.store_scatter(e_v, [seg], c_incl, mask=m_end)
        plsc.store_scatter(s_v, [seg_next], c_incl, mask=m_end)
        return lax.squeeze(lax.slice(c_incl, (L - 1,), (L,)), (0,))

    def diff_body(j, carry):
        s_v[pl.ds(j * L, L)] = e_v[pl.ds(j * L, L)] - s_v[pl.ds(j * L, L)]
        return carry

    lax.fori_loop(0, NSEG // L, diff_body, 0)

    pltpu.sync_copy(s_v.at[pl.ds(0, NSEG)], out_hbm.at[wid])


_sc_partials = pl.kernel(
    _sc_body,
    out_type=jax.ShapeDtypeStruct((NW, NSEG), jnp.float32),
    mesh=plsc.VectorSubcoreMesh(
        core_axis_name="c", subcore_axis_name="s", num_cores=2, num_subcores=16),
    compiler_params=pltpu.CompilerParams(
        needs_layout_passes=False, disable_bounds_checks=True),
    scratch_types=[
        pltpu.VMEM((96,), jnp.float32),
        pltpu.VMEM((CHUNK,), jnp.int32),
        pltpu.VMEM((CHUNK + L,), jnp.int32),
        pltpu.VMEM((SEG_PAD,), jnp.float32),
        pltpu.VMEM((SEG_PAD,), jnp.float32),
    ],
)


def _reduce_body(in_ref, out_ref):
    out_ref[...] = jnp.sum(in_ref[...], axis=0)


_tc_reduce = pl.pallas_call(
    _reduce_body,
    out_shape=jax.ShapeDtypeStruct((NSEG,), jnp.float32),
)


def kernel(property_offset, node_type, segment_ids):
    partials = _sc_partials(property_offset.astype(jnp.float32),
                            node_type.astype(jnp.int32),
                            segment_ids.astype(jnp.int32))
    return _tc_reduce(partials)


# speedup vs baseline: 14.6250x; 1.0734x over previous
import jax
import jax.numpy as jnp
from jax import lax
from jax.experimental import pallas as pl
from jax.experimental.pallas import tpu as pltpu
from jax.experimental.pallas import tpu_sc as plsc

L = 16
NW = 32
N = 100000
CHUNK = 3136
LAST = N - (NW - 1) * CHUNK
NSEG = 1024
SEG_PAD = 1040
SENT = 1032
MAX_Z = 89


def _sc_body(po_hbm, nt_hbm, seg_hbm, out_hbm, po_v, nt_v, seg_v, s_v, e_v,
             sem_po, sem_nt, sem_seg):
    c = lax.axis_index("c")
    s = lax.axis_index("s")
    wid = s * 2 + c
    base = wid * CHUNK
    is_last = wid == NW - 1
    nvec = jnp.where(is_last, LAST // L, CHUNK // L)

    pltpu.make_async_copy(po_hbm, po_v.at[pl.ds(0, MAX_Z)], sem_po).start()

    @pl.when(jnp.logical_not(is_last))
    def _():
        pltpu.make_async_copy(
            nt_hbm.at[pl.ds(base, CHUNK)], nt_v, sem_nt).start()
        pltpu.make_async_copy(
            seg_hbm.at[pl.ds(base, CHUNK)], seg_v.at[pl.ds(0, CHUNK)],
            sem_seg).start()

    @pl.when(is_last)
    def _():
        pltpu.make_async_copy(
            nt_hbm.at[pl.ds(base, LAST)], nt_v.at[pl.ds(0, LAST)],
            sem_nt).start()
        pltpu.make_async_copy(
            seg_hbm.at[pl.ds(base, LAST)], seg_v.at[pl.ds(0, LAST)],
            sem_seg).start()

    zeros = jnp.zeros((L,), jnp.float32)

    @plsc.parallel_loop(0, SEG_PAD // L, unroll=4)
    def _(j):
        s_v[pl.ds(j * L, L)] = zeros
        e_v[pl.ds(j * L, L)] = zeros

    pltpu.make_async_copy(po_hbm, po_v.at[pl.ds(0, MAX_Z)], sem_po).wait()

    @pl.when(jnp.logical_not(is_last))
    def _():
        pltpu.make_async_copy(
            nt_hbm.at[pl.ds(base, CHUNK)], nt_v, sem_nt).wait()
        pltpu.make_async_copy(
            seg_hbm.at[pl.ds(base, CHUNK)], seg_v.at[pl.ds(0, CHUNK)],
            sem_seg).wait()

    @pl.when(is_last)
    def _():
        pltpu.make_async_copy(
            nt_hbm.at[pl.ds(base, LAST)], nt_v.at[pl.ds(0, LAST)],
            sem_nt).wait()
        pltpu.make_async_copy(
            seg_hbm.at[pl.ds(base, LAST)], seg_v.at[pl.ds(0, LAST)],
            sem_seg).wait()

    seg_v[pl.ds(nvec * L, L)] = jnp.full((L,), SENT, jnp.int32)

    @plsc.parallel_loop(0, nvec, unroll=8, carry=jnp.float32(0.0))
    def _(i, carry):
        off = i * L
        seg = seg_v[pl.ds(off, L)]
        seg_next = seg_v[pl.ds(off + 1, L)]
        nt = nt_v[pl.ds(off, L)]
        v = plsc.load_gather(po_v, [nt])
        c_incl = plsc.cumsum(v) + carry
        m_end = seg != seg_next
        plsc.store_scatter(e_v, [seg], c_incl, mask=m_end)
        plsc.store_scatter(s_v, [seg_next], c_incl, mask=m_end)
        return lax.squeeze(lax.slice(c_incl, (L - 1,), (L,)), (0,))

    @plsc.parallel_loop(0, NSEG // L, unroll=4)
    def _(j):
        s_v[pl.ds(j * L, L)] = e_v[pl.ds(j * L, L)] - s_v[pl.ds(j * L, L)]

    pltpu.sync_copy(s_v.at[pl.ds(0, NSEG)], out_hbm.at[wid])


_sc_partials = pl.kernel(
    _sc_body,
    out_type=jax.ShapeDtypeStruct((NW, NSEG), jnp.float32),
    mesh=plsc.VectorSubcoreMesh(
        core_axis_name="c", subcore_axis_name="s", num_cores=2, num_subcores=16),
    compiler_params=pltpu.CompilerParams(
        needs_layout_passes=False, disable_bounds_checks=True),
    scratch_types=[
        pltpu.VMEM((96,), jnp.float32),
        pltpu.VMEM((CHUNK,), jnp.int32),
        pltpu.VMEM((CHUNK + L,), jnp.int32),
        pltpu.VMEM((SEG_PAD,), jnp.float32),
        pltpu.VMEM((SEG_PAD,), jnp.float32),
        pltpu.SemaphoreType.DMA,
        pltpu.SemaphoreType.DMA,
        pltpu.SemaphoreType.DMA,
    ],
)


def _reduce_body(in_ref, out_ref):
    out_ref[...] = jnp.sum(in_ref[...], axis=0)


_tc_reduce = pl.pallas_call(
    _reduce_body,
    out_shape=jax.ShapeDtypeStruct((NSEG,), jnp.float32),
)


def kernel(property_offset, node_type, segment_ids):
    partials = _sc_partials(property_offset.astype(jnp.float32),
                            node_type.astype(jnp.int32),
                            segment_ids.astype(jnp.int32))
    return _tc_reduce(partials)
